# baseline (speedup 1.0000x reference)
import jax
import jax.numpy as jnp
from jax import lax
from jax.experimental import pallas as pl
from jax.experimental.pallas import tpu as pltpu
from jax.experimental.pallas import tpu_sc as plsc

_EMB = 64
_L = 200
_SCALE = 8.0

_NC = 2
_NS = 16
_NW = _NC * _NS
_BB = 128


def _emb_body(tokens_hbm, table_hbm, out_hbm,
              idx_v, g0, g1, t0, t1, gsem0, gsem1, osem0, osem1):
    wid = lax.axis_index("s") * _NC + lax.axis_index("c")

    gbuf = (g0, g1)
    tbuf = (t0, t1)
    gsem = (gsem0, gsem1)
    osem = (osem0, osem1)

    pltpu.sync_copy(tokens_hbm.at[wid], idx_v)

    def dbl(i, c):
        for j in range(_BB // 16):
            sl = pl.ds(j * 16, 16)
            idx_v[i, sl] = idx_v[i, sl] * 2
        return c

    lax.fori_loop(0, _L, dbl, 0)

    def start_gather(l, p):
        pltpu.async_copy(table_hbm.at[idx_v.at[l]], gbuf[p], gsem[p])

    def wait_gather(p):
        pltpu.make_async_copy(table_hbm.at[pl.ds(0, _BB)], gbuf[p],
                              gsem[p]).wait()

    def start_out(l, p):
        for e8 in range(_EMB // 8):
            pltpu.async_copy(tbuf[p].at[pl.ds(e8 * 8, 8)],
                             out_hbm.at[l, e8, wid], osem[p])

    def wait_out(p):
        for e8 in range(_EMB // 8):
            pltpu.make_async_copy(tbuf[p].at[pl.ds(e8 * 8, 8)],
                                  out_hbm.at[0, e8, wid], osem[p]).wait()

    def transpose_scale(p):
        src, dst = gbuf[p], tbuf[p]
        rows = [lax.iota(jnp.int32, 16) + (bj * 16) for bj in range(_BB // 16)]

        def col(e, c):
            ev = jnp.full((16,), e, jnp.int32)
            for bj in range(_BB // 16):
                v = plsc.load_gather(src, [rows[bj], ev])
                dst[e, pl.ds(bj * 16, 16)] = v * _SCALE
            return c

        lax.fori_loop(0, _EMB, col, 0)

    start_gather(0, 0)

    def outer(l2, carry):
        l0 = 2 * l2
        start_gather(l0 + 1, 1)
        wait_gather(0)

        @pl.when(l2 > 0)
        def _():
            wait_out(0)
        transpose_scale(0)
        start_out(l0, 0)

        @pl.when(l2 < _L // 2 - 1)
        def _():
            start_gather(l0 + 2, 0)
        wait_gather(1)

        @pl.when(l2 > 0)
        def _():
            wait_out(1)
        transpose_scale(1)
        start_out(l0 + 1, 1)
        return carry

    lax.fori_loop(0, _L // 2, outer, 0)
    wait_out(0)
    wait_out(1)


def kernel(tokens, table):
    b, l = tokens.shape
    tokens_arr = tokens.T.reshape(l, _NW, _BB).transpose(1, 0, 2)
    table_pad = jnp.pad(table, ((0, 0), (0, 64))).reshape(-1, _EMB)
    mesh = plsc.VectorSubcoreMesh(core_axis_name="c", subcore_axis_name="s")
    out5 = pl.kernel(
        _emb_body,
        out_type=jax.ShapeDtypeStruct((l, _EMB // 8, _NW, 8, _BB),
                                      jnp.float32),
        mesh=mesh,
        scratch_types=[
            pltpu.VMEM((_L, _BB), jnp.int32),
            pltpu.VMEM((_BB, _EMB), jnp.float32),
            pltpu.VMEM((_BB, _EMB), jnp.float32),
            pltpu.VMEM((_EMB, _BB), jnp.float32),
            pltpu.VMEM((_EMB, _BB), jnp.float32),
            pltpu.SemaphoreType.DMA,
            pltpu.SemaphoreType.DMA,
            pltpu.SemaphoreType.DMA,
            pltpu.SemaphoreType.DMA,
        ],
        compiler_params=pltpu.CompilerParams(use_tc_tiling_on_sc=False,
                                             needs_layout_passes=False),
    )(tokens_arr, table_pad)
    return out5.transpose(2, 4, 0, 1, 3).reshape(b, l, _EMB)
